# baseline (speedup 1.0000x reference)
import functools

import jax
import jax.numpy as jnp
from jax import lax
from jax.experimental import pallas as pl
from jax.experimental.pallas import tpu as pltpu
from jax.experimental.pallas import tpu_sc as plsc

F = 16
G = 64
NC, NS = 2, 16
NW = NC * NS
SUB = 128
KSUB = 6
NP = 100096
RPT = NP // NS
NB = 102400
GP = 128

_MESH = dict(core_axis_name="c", subcore_axis_name="s",
             num_cores=NC, num_subcores=NS)
_SC_PARAMS = pltpu.CompilerParams(use_tc_tiling_on_sc=False)


def _deg_pass(e2d, zeros_tile, ones_sub, n_iters):
    @functools.partial(
        pl.kernel,
        out_type=jax.ShapeDtypeStruct((NC, NP, F), jnp.float32),
        mesh=plsc.VectorSubcoreMesh(**_MESH),
        compiler_params=_SC_PARAMS,
        scratch_types=[
            pltpu.VMEM((KSUB, 2, SUB), jnp.int32),
            pltpu.VMEM((KSUB, 2, SUB), jnp.int32),
            pltpu.VMEM((SUB, F), jnp.float32),
            pltpu.VMEM_SHARED((NP, F), jnp.float32),
            pltpu.SemaphoreType.DMA,
            pltpu.SemaphoreType.DMA,
        ],
    )
    def k(e_h, z_h, one_h, out_h, ev0, ev1, ones_v, acc_sh, isem, ssem):
        cid = lax.axis_index("c")
        sid = lax.axis_index("s")
        wid = sid * NC + cid
        pltpu.sync_copy(z_h, acc_sh.at[pl.ds(sid * RPT, RPT)])
        pltpu.sync_copy(one_h, ones_v)
        plsc.subcore_barrier()
        base = wid * (n_iters * KSUB)
        ev = (ev0, ev1)

        pltpu.sync_copy(e_h.at[pl.ds(base, KSUB)], ev0)
        pltpu.async_copy(e_h.at[pl.ds(base + KSUB, KSUB)], ev1, isem)

        @pl.loop(0, n_iters // 2)
        def _(ii):
            for b in range(2):
                k_it = ii * 2 + b
                evb, evn = ev[b], ev[1 - b]
                for j in range(KSUB):
                    pltpu.sync_copy(ones_v, acc_sh.at[evb.at[j, 1]], add=True)
                pltpu.make_async_copy(e_h.at[pl.ds(base, KSUB)], evn, isem).wait()
                pltpu.async_copy(
                    e_h.at[pl.ds(base + (k_it + 2) * KSUB, KSUB)], evb, isem)

        pltpu.make_async_copy(e_h.at[pl.ds(base, KSUB)], ev0, isem).wait()
        plsc.subcore_barrier()
        pltpu.sync_copy(acc_sh.at[pl.ds(sid * RPT, RPT)],
                        out_h.at[cid, pl.ds(sid * RPT, RPT)])

    return k(e2d, zeros_tile, ones_sub)


def _edge_pass(table, e2d, zeros_tile, n_iters):
    @functools.partial(
        pl.kernel,
        out_type=jax.ShapeDtypeStruct((NC, NP, F), jnp.float32),
        mesh=plsc.VectorSubcoreMesh(**_MESH),
        compiler_params=_SC_PARAMS,
        scratch_types=[
            pltpu.VMEM((KSUB, 2, SUB), jnp.int32),
            pltpu.VMEM((KSUB, 2, SUB), jnp.int32),
            pltpu.VMEM((KSUB * SUB, F), jnp.float32),
            pltpu.VMEM((KSUB * SUB, F), jnp.float32),
            pltpu.VMEM_SHARED((NP, F), jnp.float32),
            pltpu.SemaphoreType.DMA,
            pltpu.SemaphoreType.DMA,
            pltpu.SemaphoreType.DMA,
        ],
    )
    def k(tab_h, e_h, z_h, out_h, ev0, ev1, rows0, rows1, acc_sh, isem, gsem,
          ssem):
        cid = lax.axis_index("c")
        sid = lax.axis_index("s")
        wid = sid * NC + cid
        pltpu.sync_copy(z_h, acc_sh.at[pl.ds(sid * RPT, RPT)])
        plsc.subcore_barrier()
        base = wid * (n_iters * KSUB)
        ev = (ev0, ev1)
        rows = (rows0, rows1)

        def fire_gathers(evx, rowsx):
            for j in range(KSUB):
                pltpu.async_copy(tab_h.at[evx.at[j, 0]],
                                 rowsx.at[pl.ds(j * SUB, SUB)], gsem)

        def drain_gathers(rowsx):
            for j in range(KSUB):
                pltpu.make_async_copy(
                    tab_h.at[pl.ds(0, SUB)],
                    rowsx.at[pl.ds(j * SUB, SUB)], gsem).wait()

        pltpu.sync_copy(e_h.at[pl.ds(base, KSUB)], ev0)
        pltpu.async_copy(e_h.at[pl.ds(base + KSUB, KSUB)], ev1, isem)
        fire_gathers(ev0, rows0)

        @pl.loop(0, n_iters // 2)
        def _(ii):
            for b in range(2):
                k_it = ii * 2 + b
                evb, evn = ev[b], ev[1 - b]
                rowsb, rowsn = rows[b], rows[1 - b]
                drain_gathers(rowsb)
                pltpu.make_async_copy(e_h.at[pl.ds(base, KSUB)], evn, isem).wait()
                fire_gathers(evn, rowsn)
                for j in range(KSUB):
                    pltpu.sync_copy(rowsb.at[pl.ds(j * SUB, SUB)],
                                    acc_sh.at[evb.at[j, 1]], add=True)
                pltpu.async_copy(
                    e_h.at[pl.ds(base + (k_it + 2) * KSUB, KSUB)], evb, isem)

        drain_gathers(rows0)
        pltpu.make_async_copy(e_h.at[pl.ds(base, KSUB)], ev0, isem).wait()
        plsc.subcore_barrier()
        pltpu.sync_copy(acc_sh.at[pl.ds(sid * RPT, RPT)],
                        out_h.at[cid, pl.ds(sid * RPT, RPT)])

    return k(table, e2d, zeros_tile)


def _pool_pass(w_nb, batch2d, zeros_gp, ones_sub, n_iters):
    @functools.partial(
        pl.kernel,
        out_type=jax.ShapeDtypeStruct((NC, 2, GP, F), jnp.float32),
        mesh=plsc.VectorSubcoreMesh(**_MESH),
        compiler_params=_SC_PARAMS,
        scratch_types=[
            pltpu.VMEM((1, SUB), jnp.int32),
            pltpu.VMEM((SUB, F), jnp.float32),
            pltpu.VMEM((SUB, F), jnp.float32),
            pltpu.VMEM_SHARED((GP, F), jnp.float32),
            pltpu.VMEM_SHARED((GP, F), jnp.float32),
        ],
    )
    def k(w_h, b_h, z_h, one_h, out_h, b_v, w_v, ones_v, qacc, cacc):
        cid = lax.axis_index("c")
        sid = lax.axis_index("s")
        wid = sid * NC + cid

        @pl.when(sid == 0)
        def _():
            pltpu.sync_copy(z_h, qacc)
            pltpu.sync_copy(z_h, cacc)

        pltpu.sync_copy(one_h, ones_v)
        plsc.subcore_barrier()
        base = wid * n_iters

        @pl.loop(0, n_iters)
        def _(i):
            r = base + i
            pltpu.sync_copy(b_h.at[pl.ds(r, 1)], b_v)
            pltpu.sync_copy(w_h.at[pl.ds(r * SUB, SUB)], w_v)
            pltpu.sync_copy(w_v, qacc.at[b_v.at[0]], add=True)
            pltpu.sync_copy(ones_v, cacc.at[b_v.at[0]], add=True)

        plsc.subcore_barrier()

        @pl.when(sid == 0)
        def _():
            pltpu.sync_copy(qacc, out_h.at[cid, 0])
            pltpu.sync_copy(cacc, out_h.at[cid, 1])

    return k(w_nb, batch2d, zeros_gp, ones_sub)


NF = NP * F // 128
BNF = NF // 4
BN1 = NP // 32


def _dense1a(degp_flat):
    def body(d0, d1, dinv_o):
        dinv_o[...] = lax.rsqrt(d0[...] + d1[...] + 1.0)

    return pl.pallas_call(
        body,
        grid=(NF // BNF,),
        in_specs=[
            pl.BlockSpec((BNF, 128), lambda i: (i, 0)),
            pl.BlockSpec((BNF, 128), lambda i: (NF // BNF + i, 0)),
        ],
        out_specs=pl.BlockSpec((BNF, 128), lambda i: (i, 0)),
        out_shape=jax.ShapeDtypeStruct((NF, 128), jnp.float32),
    )(degp_flat, degp_flat)


def _dense1b(dinv16, x_p, W1):
    def body(dv, xr, w1, xw1s_o):
        xb = xr[...].astype(jnp.bfloat16)
        w1b = w1[...].astype(jnp.bfloat16)
        xw1 = jnp.dot(xb, w1b, preferred_element_type=jnp.float32)
        xw1s_o[...] = dv[...] * xw1

    return pl.pallas_call(
        body,
        grid=(NP // BN1,),
        in_specs=[
            pl.BlockSpec((BN1, F), lambda i: (i, 0)),
            pl.BlockSpec((BN1, 3), lambda i: (i, 0)),
            pl.BlockSpec((3, F), lambda i: (0, 0)),
        ],
        out_specs=pl.BlockSpec((BN1, F), lambda i: (i, 0)),
        out_shape=jax.ShapeDtypeStruct((NP, F), jnp.float32),
    )(dinv16, x_p, W1)


def _dense2(up_flat, xw1s_flat, dinv_flat, b1_flat, W2blk):
    def body(u0, u1, xs, dv, b1, w2, hws_o):
        dinv = dv[...]
        pre = dinv * (u0[...] + u1[...] + xs[...]) + b1[...]
        h = jnp.maximum(pre, 0.0)
        hw = jnp.dot(h.astype(jnp.bfloat16), w2[...],
                     preferred_element_type=jnp.float32)
        hws_o[...] = dinv * hw

    return pl.pallas_call(
        body,
        grid=(NF // BNF,),
        in_specs=[
            pl.BlockSpec((BNF, 128), lambda i: (i, 0)),
            pl.BlockSpec((BNF, 128), lambda i: (NF // BNF + i, 0)),
            pl.BlockSpec((BNF, 128), lambda i: (i, 0)),
            pl.BlockSpec((BNF, 128), lambda i: (i, 0)),
            pl.BlockSpec((1, 128), lambda i: (0, 0)),
            pl.BlockSpec((128, 128), lambda i: (0, 0)),
        ],
        out_specs=pl.BlockSpec((BNF, 128), lambda i: (i, 0)),
        out_shape=jax.ShapeDtypeStruct((NF, 128), jnp.float32),
    )(up_flat, up_flat, xw1s_flat, dinv_flat, b1_flat, W2blk)


def _dense3(vp_flat, hws_flat, dinv_flat):
    def body(v0, v1, hv, dv, w_o):
        w_o[...] = dv[...] * (v0[...] + v1[...] + hv[...])

    return pl.pallas_call(
        body,
        grid=(NF // BNF,),
        in_specs=[
            pl.BlockSpec((BNF, 128), lambda i: (i, 0)),
            pl.BlockSpec((BNF, 128), lambda i: (NF // BNF + i, 0)),
            pl.BlockSpec((BNF, 128), lambda i: (i, 0)),
            pl.BlockSpec((BNF, 128), lambda i: (i, 0)),
        ],
        out_specs=pl.BlockSpec((BNF, 128), lambda i: (i, 0)),
        out_shape=jax.ShapeDtypeStruct((NF, 128), jnp.float32),
    )(vp_flat, vp_flat, hws_flat, dinv_flat)


def _dense_head(pool, b2r, fW1, fb1r, fW2, fb2r):
    def body(p_ref, b2, f1, g1, f2, g2, o_ref):
        pv = p_ref[...]
        q = pv[0, 0, :G, :] + pv[1, 0, :G, :]
        cnt = pv[0, 1, :G, 0:1] + pv[1, 1, :G, 0:1]
        p = q + cnt * b2[...]
        pb = p.astype(jnp.bfloat16).astype(jnp.float32)
        f1v = f1[...].astype(jnp.bfloat16).astype(jnp.float32)
        acc = jnp.broadcast_to(g1[...], (G, F))
        for kk in range(F):
            acc = acc + pb[:, kk:kk + 1] * f1v[kk:kk + 1, :]
        zrelu = jnp.maximum(acc, 0.0)
        zb = zrelu.astype(jnp.bfloat16).astype(jnp.float32)
        f2v = f2[...].astype(jnp.bfloat16).astype(jnp.float32)
        o = jnp.broadcast_to(g2[...], (G, 7))
        for kk in range(F):
            o = o + zb[:, kk:kk + 1] * f2v[kk:kk + 1, :]
        m = jnp.max(o, axis=1, keepdims=True)
        s = jnp.log(jnp.sum(jnp.exp(o - m), axis=1, keepdims=True))
        o_ref[...] = o - m - s

    return pl.pallas_call(
        body,
        in_specs=[
            pl.BlockSpec((NC, 2, GP, F), lambda: (0, 0, 0, 0)),
            pl.BlockSpec((1, F), lambda: (0, 0)),
            pl.BlockSpec((F, F), lambda: (0, 0)),
            pl.BlockSpec((1, F), lambda: (0, 0)),
            pl.BlockSpec((F, 7), lambda: (0, 0)),
            pl.BlockSpec((1, 7), lambda: (0, 0)),
        ],
        out_specs=pl.BlockSpec((G, 7), lambda: (0, 0)),
        out_shape=jax.ShapeDtypeStruct((G, 7), jnp.float32),
    )(pool, b2r, fW1, fb1r, fW2, fb2r)


def kernel(x, edge_index, batch, W1, b1, W2, b2, fW1, fb1, fW2, fb2):
    N = x.shape[0]
    E = edge_index.shape[1]
    chunk = 2 * NW * SUB * KSUB
    EP = -(-E // chunk) * chunk
    n_iters = EP // (NW * SUB * KSUB)
    pad = EP - E

    src_p = jnp.concatenate(
        [edge_index[0], jnp.zeros((pad,), edge_index.dtype)]
    ).reshape(EP // SUB, SUB)
    dst_p = jnp.concatenate(
        [edge_index[1], jnp.full((pad,), NP - 1, edge_index.dtype)]
    ).reshape(EP // SUB, SUB)
    e2d = jnp.pad(jnp.stack([src_p, dst_p], axis=1),
                  ((0, 2 * KSUB), (0, 0), (0, 0)))
    x_p = jnp.pad(x, ((0, NP - N), (0, 0)))
    zeros_tile = jnp.zeros((RPT, F), jnp.float32)
    ones_sub = jnp.ones((SUB, F), jnp.float32)
    zeros_gp = jnp.zeros((GP, F), jnp.float32)

    b1_flat = jnp.tile(b1, 8).reshape(1, 128)
    W2blk = jnp.kron(jnp.eye(8, dtype=jnp.float32), W2).astype(jnp.bfloat16)

    degp = _deg_pass(e2d, zeros_tile, ones_sub, n_iters)
    dinv_flat = _dense1a(degp.reshape(2 * NF, 128))
    xw1s = _dense1b(dinv_flat.reshape(NP, F), x_p, W1)
    up = _edge_pass(xw1s, e2d, zeros_tile, n_iters)
    hws_flat = _dense2(up.reshape(2 * NF, 128), xw1s.reshape(NF, 128),
                       dinv_flat, b1_flat, W2blk)
    vp = _edge_pass(hws_flat.reshape(NP, F), e2d, zeros_tile, n_iters)
    w_flat = _dense3(vp.reshape(2 * NF, 128), hws_flat, dinv_flat)
    w = w_flat.reshape(NP, F)

    w_nb = jnp.pad(w[:N], ((0, NB - N), (0, 0)))
    batch_nb = jnp.pad(batch, (0, NB - N),
                       constant_values=G).reshape(NB // SUB, SUB)
    pool = _pool_pass(w_nb, batch_nb, zeros_gp, ones_sub, NB // (NW * SUB))

    return _dense_head(pool, b2.reshape(1, F), fW1, fb1.reshape(1, F),
                       fW2, fb2.reshape(1, 7))


# speedup vs baseline: 1.1640x; 1.1640x over previous
import functools

import jax
import jax.numpy as jnp
from jax import lax
from jax.experimental import pallas as pl
from jax.experimental.pallas import tpu as pltpu
from jax.experimental.pallas import tpu_sc as plsc

F = 16
G = 64
NC, NS = 2, 16
NW = NC * NS
SUB = 128
KSUB = 4
NP = 100096
RPT = NP // NS
NB = 102400
GP = 128

_MESH = dict(core_axis_name="c", subcore_axis_name="s",
             num_cores=NC, num_subcores=NS)
_SC_PARAMS = pltpu.CompilerParams(use_tc_tiling_on_sc=False)


def _deg_pass(e2d, zeros_tile, ones_sub, n_iters):
    @functools.partial(
        pl.kernel,
        out_type=jax.ShapeDtypeStruct((NC, NP, F), jnp.float32),
        mesh=plsc.VectorSubcoreMesh(**_MESH),
        compiler_params=_SC_PARAMS,
        scratch_types=[
            pltpu.VMEM((KSUB, 2, SUB), jnp.int32),
            pltpu.VMEM((KSUB, 2, SUB), jnp.int32),
            pltpu.VMEM((SUB, F), jnp.float32),
            pltpu.VMEM_SHARED((NP, F), jnp.float32),
            pltpu.SemaphoreType.DMA,
        ],
    )
    def k(e_h, z_h, one_h, out_h, ev0, ev1, ones_v, acc_sh, isem):
        cid = lax.axis_index("c")
        sid = lax.axis_index("s")
        wid = sid * NC + cid
        pltpu.sync_copy(z_h, acc_sh.at[pl.ds(sid * RPT, RPT)])
        pltpu.sync_copy(one_h, ones_v)
        plsc.subcore_barrier()
        base = wid * (n_iters * KSUB)
        ev = (ev0, ev1)

        pltpu.sync_copy(e_h.at[pl.ds(base, KSUB)], ev0)
        pltpu.async_copy(e_h.at[pl.ds(base + KSUB, KSUB)], ev1, isem)

        @pl.loop(0, n_iters // 2)
        def _(ii):
            for b in range(2):
                k_it = ii * 2 + b
                evb, evn = ev[b], ev[1 - b]
                for j in range(KSUB):
                    pltpu.sync_copy(ones_v, acc_sh.at[evb.at[j, 1]], add=True)
                pltpu.make_async_copy(e_h.at[pl.ds(base, KSUB)], evn, isem).wait()
                pltpu.async_copy(
                    e_h.at[pl.ds(base + (k_it + 2) * KSUB, KSUB)], evb, isem)

        pltpu.make_async_copy(e_h.at[pl.ds(base, KSUB)], ev0, isem).wait()
        plsc.subcore_barrier()
        pltpu.sync_copy(acc_sh.at[pl.ds(sid * RPT, RPT)],
                        out_h.at[cid, pl.ds(sid * RPT, RPT)])

    return k(e2d, zeros_tile, ones_sub)


def _edge_pass(table, e2d, zeros_tile, n_iters):
    @functools.partial(
        pl.kernel,
        out_type=jax.ShapeDtypeStruct((NC, NP, F), jnp.float32),
        mesh=plsc.VectorSubcoreMesh(**_MESH),
        compiler_params=_SC_PARAMS,
        scratch_types=[
            pltpu.VMEM((KSUB, 2, SUB), jnp.int32),
            pltpu.VMEM((KSUB, 2, SUB), jnp.int32),
            pltpu.VMEM((KSUB * SUB, F), jnp.float32),
            pltpu.VMEM((KSUB * SUB, F), jnp.float32),
            pltpu.VMEM_SHARED((NP, F), jnp.float32),
            pltpu.SemaphoreType.DMA,
            pltpu.SemaphoreType.DMA,
        ],
    )
    def k(tab_h, e_h, z_h, out_h, ev0, ev1, rows0, rows1, acc_sh, isem, gsem):
        cid = lax.axis_index("c")
        sid = lax.axis_index("s")
        wid = sid * NC + cid
        pltpu.sync_copy(z_h, acc_sh.at[pl.ds(sid * RPT, RPT)])
        plsc.subcore_barrier()
        base = wid * (n_iters * KSUB)
        ev = (ev0, ev1)
        rows = (rows0, rows1)

        def fire_gathers(evx, rowsx):
            for j in range(KSUB):
                pltpu.async_copy(tab_h.at[evx.at[j, 0]],
                                 rowsx.at[pl.ds(j * SUB, SUB)], gsem)

        def drain_gathers(rowsx):
            for j in range(KSUB):
                pltpu.make_async_copy(
                    tab_h.at[pl.ds(0, SUB)],
                    rowsx.at[pl.ds(j * SUB, SUB)], gsem).wait()

        pltpu.sync_copy(e_h.at[pl.ds(base, KSUB)], ev0)
        pltpu.async_copy(e_h.at[pl.ds(base + KSUB, KSUB)], ev1, isem)
        fire_gathers(ev0, rows0)

        @pl.loop(0, n_iters // 2)
        def _(ii):
            for b in range(2):
                k_it = ii * 2 + b
                evb, evn = ev[b], ev[1 - b]
                rowsb, rowsn = rows[b], rows[1 - b]
                drain_gathers(rowsb)
                pltpu.make_async_copy(e_h.at[pl.ds(base, KSUB)], evn, isem).wait()
                fire_gathers(evn, rowsn)
                for j in range(KSUB):
                    pltpu.sync_copy(rowsb.at[pl.ds(j * SUB, SUB)],
                                    acc_sh.at[evb.at[j, 1]], add=True)
                pltpu.async_copy(
                    e_h.at[pl.ds(base + (k_it + 2) * KSUB, KSUB)], evb, isem)

        drain_gathers(rows0)
        pltpu.make_async_copy(e_h.at[pl.ds(base, KSUB)], ev0, isem).wait()
        plsc.subcore_barrier()
        pltpu.sync_copy(acc_sh.at[pl.ds(sid * RPT, RPT)],
                        out_h.at[cid, pl.ds(sid * RPT, RPT)])

    return k(table, e2d, zeros_tile)


def _pool_pass(w_nb, batch2d, zeros_gp, ones_sub, n_iters):
    @functools.partial(
        pl.kernel,
        out_type=jax.ShapeDtypeStruct((NC, 2, GP, F), jnp.float32),
        mesh=plsc.VectorSubcoreMesh(**_MESH),
        compiler_params=_SC_PARAMS,
        scratch_types=[
            pltpu.VMEM((1, SUB), jnp.int32),
            pltpu.VMEM((SUB, F), jnp.float32),
            pltpu.VMEM((SUB, F), jnp.float32),
            pltpu.VMEM_SHARED((GP, F), jnp.float32),
            pltpu.VMEM_SHARED((GP, F), jnp.float32),
        ],
    )
    def k(w_h, b_h, z_h, one_h, out_h, b_v, w_v, ones_v, qacc, cacc):
        cid = lax.axis_index("c")
        sid = lax.axis_index("s")
        wid = sid * NC + cid

        @pl.when(sid == 0)
        def _():
            pltpu.sync_copy(z_h, qacc)
            pltpu.sync_copy(z_h, cacc)

        pltpu.sync_copy(one_h, ones_v)
        plsc.subcore_barrier()
        base = wid * n_iters

        @pl.loop(0, n_iters)
        def _(i):
            r = base + i
            pltpu.sync_copy(b_h.at[pl.ds(r, 1)], b_v)
            pltpu.sync_copy(w_h.at[pl.ds(r * SUB, SUB)], w_v)
            pltpu.sync_copy(w_v, qacc.at[b_v.at[0]], add=True)
            pltpu.sync_copy(ones_v, cacc.at[b_v.at[0]], add=True)

        plsc.subcore_barrier()

        @pl.when(sid == 0)
        def _():
            pltpu.sync_copy(qacc, out_h.at[cid, 0])
            pltpu.sync_copy(cacc, out_h.at[cid, 1])

    return k(w_nb, batch2d, zeros_gp, ones_sub)


NF = NP * F // 128
BNF = NF // 4
BN1 = NP // 32


def _dense1a(degp_flat):
    def body(d0, d1, dinv_o):
        dinv_o[...] = lax.rsqrt(d0[...] + d1[...] + 1.0)

    return pl.pallas_call(
        body,
        grid=(NF // BNF,),
        in_specs=[
            pl.BlockSpec((BNF, 128), lambda i: (i, 0)),
            pl.BlockSpec((BNF, 128), lambda i: (NF // BNF + i, 0)),
        ],
        out_specs=pl.BlockSpec((BNF, 128), lambda i: (i, 0)),
        out_shape=jax.ShapeDtypeStruct((NF, 128), jnp.float32),
    )(degp_flat, degp_flat)


def _dense1b(dinv16, x_p, W1):
    def body(dv, xr, w1, xw1s_o):
        xb = xr[...].astype(jnp.bfloat16)
        w1b = w1[...].astype(jnp.bfloat16)
        xw1 = jnp.dot(xb, w1b, preferred_element_type=jnp.float32)
        xw1s_o[...] = dv[...] * xw1

    return pl.pallas_call(
        body,
        grid=(NP // BN1,),
        in_specs=[
            pl.BlockSpec((BN1, F), lambda i: (i, 0)),
            pl.BlockSpec((BN1, 3), lambda i: (i, 0)),
            pl.BlockSpec((3, F), lambda i: (0, 0)),
        ],
        out_specs=pl.BlockSpec((BN1, F), lambda i: (i, 0)),
        out_shape=jax.ShapeDtypeStruct((NP, F), jnp.float32),
    )(dinv16, x_p, W1)


def _dense2(up_flat, xw1s_flat, dinv_flat, b1_flat, W2blk):
    def body(u0, u1, xs, dv, b1, w2, hws_o):
        dinv = dv[...]
        pre = dinv * (u0[...] + u1[...] + xs[...]) + b1[...]
        h = jnp.maximum(pre, 0.0)
        hw = jnp.dot(h.astype(jnp.bfloat16), w2[...],
                     preferred_element_type=jnp.float32)
        hws_o[...] = dinv * hw

    return pl.pallas_call(
        body,
        grid=(NF // BNF,),
        in_specs=[
            pl.BlockSpec((BNF, 128), lambda i: (i, 0)),
            pl.BlockSpec((BNF, 128), lambda i: (NF // BNF + i, 0)),
            pl.BlockSpec((BNF, 128), lambda i: (i, 0)),
            pl.BlockSpec((BNF, 128), lambda i: (i, 0)),
            pl.BlockSpec((1, 128), lambda i: (0, 0)),
            pl.BlockSpec((128, 128), lambda i: (0, 0)),
        ],
        out_specs=pl.BlockSpec((BNF, 128), lambda i: (i, 0)),
        out_shape=jax.ShapeDtypeStruct((NF, 128), jnp.float32),
    )(up_flat, up_flat, xw1s_flat, dinv_flat, b1_flat, W2blk)


def _dense3(vp_flat, hws_flat, dinv_flat):
    def body(v0, v1, hv, dv, w_o):
        w_o[...] = dv[...] * (v0[...] + v1[...] + hv[...])

    return pl.pallas_call(
        body,
        grid=(NF // BNF,),
        in_specs=[
            pl.BlockSpec((BNF, 128), lambda i: (i, 0)),
            pl.BlockSpec((BNF, 128), lambda i: (NF // BNF + i, 0)),
            pl.BlockSpec((BNF, 128), lambda i: (i, 0)),
            pl.BlockSpec((BNF, 128), lambda i: (i, 0)),
        ],
        out_specs=pl.BlockSpec((BNF, 128), lambda i: (i, 0)),
        out_shape=jax.ShapeDtypeStruct((NF, 128), jnp.float32),
    )(vp_flat, vp_flat, hws_flat, dinv_flat)


def _dense_head(pool, b2r, fW1, fb1r, fW2, fb2r):
    def body(p_ref, b2, f1, g1, f2, g2, o_ref):
        pv = p_ref[...]
        q = pv[0, 0, :G, :] + pv[1, 0, :G, :]
        cnt = pv[0, 1, :G, 0:1] + pv[1, 1, :G, 0:1]
        p = q + cnt * b2[...]
        pb = p.astype(jnp.bfloat16).astype(jnp.float32)
        f1v = f1[...].astype(jnp.bfloat16).astype(jnp.float32)
        acc = jnp.broadcast_to(g1[...], (G, F))
        for kk in range(F):
            acc = acc + pb[:, kk:kk + 1] * f1v[kk:kk + 1, :]
        zrelu = jnp.maximum(acc, 0.0)
        zb = zrelu.astype(jnp.bfloat16).astype(jnp.float32)
        f2v = f2[...].astype(jnp.bfloat16).astype(jnp.float32)
        o = jnp.broadcast_to(g2[...], (G, 7))
        for kk in range(F):
            o = o + zb[:, kk:kk + 1] * f2v[kk:kk + 1, :]
        m = jnp.max(o, axis=1, keepdims=True)
        s = jnp.log(jnp.sum(jnp.exp(o - m), axis=1, keepdims=True))
        o_ref[...] = o - m - s

    return pl.pallas_call(
        body,
        in_specs=[
            pl.BlockSpec((NC, 2, GP, F), lambda: (0, 0, 0, 0)),
            pl.BlockSpec((1, F), lambda: (0, 0)),
            pl.BlockSpec((F, F), lambda: (0, 0)),
            pl.BlockSpec((1, F), lambda: (0, 0)),
            pl.BlockSpec((F, 7), lambda: (0, 0)),
            pl.BlockSpec((1, 7), lambda: (0, 0)),
        ],
        out_specs=pl.BlockSpec((G, 7), lambda: (0, 0)),
        out_shape=jax.ShapeDtypeStruct((G, 7), jnp.float32),
    )(pool, b2r, fW1, fb1r, fW2, fb2r)


def kernel(x, edge_index, batch, W1, b1, W2, b2, fW1, fb1, fW2, fb2):
    N = x.shape[0]
    E = edge_index.shape[1]
    chunk = 2 * NW * SUB * KSUB
    EP = -(-E // chunk) * chunk
    n_iters = EP // (NW * SUB * KSUB)
    pad = EP - E

    src_p = jnp.concatenate(
        [edge_index[0], jnp.zeros((pad,), edge_index.dtype)]
    ).reshape(EP // SUB, SUB)
    dst_p = jnp.concatenate(
        [edge_index[1], jnp.full((pad,), NP - 1, edge_index.dtype)]
    ).reshape(EP // SUB, SUB)
    e2d = jnp.pad(jnp.stack([src_p, dst_p], axis=1),
                  ((0, 2 * KSUB), (0, 0), (0, 0)))
    x_p = jnp.pad(x, ((0, NP - N), (0, 0)))
    zeros_tile = jnp.zeros((RPT, F), jnp.float32)
    ones_sub = jnp.ones((SUB, F), jnp.float32)
    zeros_gp = jnp.zeros((GP, F), jnp.float32)

    b1_flat = jnp.tile(b1, 8).reshape(1, 128)
    W2blk = jnp.kron(jnp.eye(8, dtype=jnp.float32), W2).astype(jnp.bfloat16)

    degp = _deg_pass(e2d, zeros_tile, ones_sub, n_iters)
    dinv_flat = _dense1a(degp.reshape(2 * NF, 128))
    xw1s = _dense1b(dinv_flat.reshape(NP, F), x_p, W1)
    up = _edge_pass(xw1s, e2d, zeros_tile, n_iters)
    hws_flat = _dense2(up.reshape(2 * NF, 128), xw1s.reshape(NF, 128),
                       dinv_flat, b1_flat, W2blk)
    vp = _edge_pass(hws_flat.reshape(NP, F), e2d, zeros_tile, n_iters)
    w_flat = _dense3(vp.reshape(2 * NF, 128), hws_flat, dinv_flat)
    w = w_flat.reshape(NP, F)

    w_nb = jnp.pad(w[:N], ((0, NB - N), (0, 0)))
    batch_nb = jnp.pad(batch, (0, NB - N),
                       constant_values=G).reshape(NB // SUB, SUB)
    pool = _pool_pass(w_nb, batch_nb, zeros_gp, ones_sub, NB // (NW * SUB))

    return _dense_head(pool, b2.reshape(1, F), fW1, fb1.reshape(1, F),
                       fW2, fb2.reshape(1, 7))
